# baseline (speedup 1.0000x reference)
import functools

import jax
import jax.numpy as jnp
from jax import lax
from jax.experimental import pallas as pl
from jax.experimental.pallas import tpu as pltpu


def _mlp_body(pooled_ref, p1_ref, pb1_ref, p2_ref, pb2_ref, p3_ref, pb3_ref,
              out_ref):
    z = jnp.maximum(
        jnp.dot(pooled_ref[...], p1_ref[...],
                preferred_element_type=jnp.float32) + pb1_ref[...], 0.0)
    z = jnp.dot(z, p2_ref[...], preferred_element_type=jnp.float32) + pb2_ref[...]
    z = jnp.dot(z, p3_ref[...], preferred_element_type=jnp.float32) + pb3_ref[...]
    out_ref[...] = z


def _mlp_head(pooled, P1, pb1, P2, pb2, P3, pb3):
    G = pooled.shape[0]
    return pl.pallas_call(
        _mlp_body,
        out_shape=jax.ShapeDtypeStruct((G, 300), jnp.float32),
    )(pooled, P1, pb1[None, :], P2, pb2[None, :], P3, pb3[None, :])


def _batchnorm(h, g, b):
    m = jnp.mean(h, axis=0)
    v = jnp.var(h, axis=0)
    return (h - m) / jnp.sqrt(v + 1e-5) * g + b


def _emp(e, esrc, edst, erel, W, Ws, b):
    msg = jnp.zeros((esrc.shape[0], W.shape[2]), e.dtype)
    for r in range(W.shape[0]):
        eW = e @ W[r]
        msg = msg + jnp.where((erel == r)[:, None], eW[esrc], 0.0)
    agg = jax.ops.segment_sum(msg, edst, num_segments=e.shape[0])
    return jax.nn.relu(agg + e @ Ws + b)


def _rgcn(h, src, dst, etype, emsg, W, Wr, We, b):
    n = h.shape[0]
    hW = jnp.einsum('nd,rdf->nrf', h, W)
    msg = hW[src, etype]
    agg = jax.ops.segment_sum(msg, dst, num_segments=n)
    agg_e = jax.ops.segment_sum(emsg, dst, num_segments=n) @ We
    return agg + agg_e + h @ Wr + b


def kernel(x, edge_attr, params, edge_index, edge_type, edge_message_index,
           edge_message_relation, batch):
    src, dst = edge_index[0], edge_index[1]
    esrc, edst = edge_message_index[0], edge_message_index[1]
    bns = ['bn1', 'bn2', 'bn3', 'bn3', 'bn3', 'bn2']
    G = 32
    e = edge_attr
    h = x
    for i in range(6):
        e = _emp(e, esrc, edst, edge_message_relation,
                 params['emp%d_W' % (i + 1)], params['emp%d_Ws' % (i + 1)],
                 params['emp%d_b' % (i + 1)])
        h = _rgcn(h, src, dst, edge_type, e,
                  params['conv%d_W' % (i + 1)], params['conv%d_Wr' % (i + 1)],
                  params['conv%d_We' % (i + 1)], params['conv%d_b' % (i + 1)])
        h = _batchnorm(h, params[bns[i] + '_g'], params[bns[i] + '_beta'])
        if i < 5:
            h = jax.nn.relu(h)
    counts = jax.ops.segment_sum(jnp.ones((h.shape[0],), jnp.float32), batch,
                                 num_segments=G)
    pooled = jax.ops.segment_sum(h, batch, num_segments=G) / \
        jnp.clip(counts, 1.0)[:, None]
    return _mlp_head(pooled, params['P1'], params['pb1'], params['P2'],
                     params['pb2'], params['P3'], params['pb3'])


# speedup vs baseline: 1.3058x; 1.3058x over previous
import functools

import jax
import jax.numpy as jnp
from jax import lax
from jax.experimental import pallas as pl
from jax.experimental.pallas import tpu as pltpu

N = 10000
E = 160000
E2 = 320000
R_NODE = 7
R_EDGE = 8
G = 32
H = 512
TILE = 256
MM_TILE = 256


def _ceil_to(x, m):
    return (x + m - 1) // m * m


def _build_emp_layout(rel, esrc, edst):
    M = rel.shape[0]
    Mp = M + R_EDGE * TILE
    perm = jnp.argsort(rel)
    cnt = jnp.bincount(rel, length=R_EDGE)
    offs = jnp.concatenate([jnp.zeros(1, jnp.int32),
                            jnp.cumsum(cnt)[:-1].astype(jnp.int32)])
    cnt_pad = (cnt + TILE - 1) // TILE * TILE
    offs_pad = jnp.concatenate([jnp.zeros(1, jnp.int32),
                                jnp.cumsum(cnt_pad)[:-1].astype(jnp.int32)])
    r_sorted = rel[perm]
    rank = jnp.arange(M, dtype=jnp.int32) - offs[r_sorted]
    pos = (offs_pad[r_sorted] + rank).astype(jnp.int32)
    gidx = jnp.zeros(Mp, jnp.int32).at[pos].set(esrc[perm].astype(jnp.int32))
    bnd = jnp.cumsum(cnt_pad).astype(jnp.int32)
    tstart = jnp.arange(Mp // TILE, dtype=jnp.int32) * TILE
    tile_rel = jnp.clip(jnp.searchsorted(bnd, tstart, side='right'),
                        0, R_EDGE - 1).astype(jnp.int32)
    pos_of_edge = jnp.zeros(M, jnp.int32).at[perm].set(pos)
    permD = jnp.argsort(edst)
    gidxD = pos_of_edge[permD]
    dstD = edst[permD].astype(jnp.int32)
    return gidx, tile_rel, gidxD, dstD, Mp


def _bucket_mm_body(tile_rel_ref, g_ref, w_ref, o_ref):
    r = tile_rel_ref[pl.program_id(0)]
    o_ref[...] = jnp.dot(g_ref[...], w_ref[r],
                         preferred_element_type=jnp.float32)


def _bucket_mm(g, w, tile_rel):
    Mp, K = g.shape
    F = w.shape[2]
    nt = Mp // TILE
    grid_spec = pltpu.PrefetchScalarGridSpec(
        num_scalar_prefetch=1,
        grid=(nt,),
        in_specs=[
            pl.BlockSpec((TILE, K), lambda t, sref: (t, 0)),
            pl.BlockSpec((R_EDGE, K, F), lambda t, sref: (0, 0, 0)),
        ],
        out_specs=pl.BlockSpec((TILE, F), lambda t, sref: (t, 0)),
    )
    return pl.pallas_call(
        _bucket_mm_body,
        grid_spec=grid_spec,
        out_shape=jax.ShapeDtypeStruct((Mp, F), jnp.float32),
    )(tile_rel, g, w)


def _fused_mm_body(nxy, relu_x, relu_out, *refs):
    if nxy == 2:
        x_ref, a_ref, y_ref, b_ref, c_ref, bias_ref, xs_ref, ys_ref, o_ref = refs
    else:
        x_ref, a_ref, bias_ref, xs_ref, o_ref = refs
        y_ref = b_ref = c_ref = ys_ref = None
    x = x_ref[...]
    xs = xs_ref[...]
    x = (x - xs[0:1, :x.shape[1]]) * xs[1:2, :x.shape[1]] + xs[2:3, :x.shape[1]]
    if relu_x:
        x = jnp.maximum(x, 0.0)
    acc = jnp.dot(x, a_ref[...], preferred_element_type=jnp.float32)
    if nxy == 2:
        y = y_ref[...]
        ys = ys_ref[...]
        y = (y - ys[0:1, :y.shape[1]]) * ys[1:2, :y.shape[1]] + ys[2:3, :y.shape[1]]
        if relu_x:
            y = jnp.maximum(y, 0.0)
        acc = acc + jnp.dot(y, b_ref[...], preferred_element_type=jnp.float32)
        acc = acc + c_ref[...]
    acc = acc + bias_ref[...]
    if relu_out:
        acc = jnp.maximum(acc, 0.0)
    o_ref[...] = acc


def _id_stats(K):
    return jnp.stack([jnp.zeros((K,), jnp.float32),
                      jnp.ones((K,), jnp.float32),
                      jnp.zeros((K,), jnp.float32)])


def _fused_mm(x, a, bias, xstats=None, y=None, b=None, c=None, ystats=None,
              relu_x=False, relu_out=False):
    M, K = x.shape
    F = a.shape[1]
    nt = _ceil_to(M, MM_TILE) // MM_TILE
    if xstats is None:
        xstats = _id_stats(K)
    nxy = 2 if y is not None else 1
    in_specs = [
        pl.BlockSpec((MM_TILE, K), lambda t: (t, 0)),
        pl.BlockSpec((K, F), lambda t: (0, 0)),
    ]
    args = [x, a]
    if nxy == 2:
        K2 = y.shape[1]
        if ystats is None:
            ystats = _id_stats(K2)
        in_specs += [
            pl.BlockSpec((MM_TILE, K2), lambda t: (t, 0)),
            pl.BlockSpec((K2, F), lambda t: (0, 0)),
            pl.BlockSpec((MM_TILE, F), lambda t: (t, 0)),
        ]
        args += [y, b, c]
    in_specs.append(pl.BlockSpec((1, F), lambda t: (0, 0)))
    args.append(bias[None, :])
    in_specs.append(pl.BlockSpec((3, K), lambda t: (0, 0)))
    args.append(xstats)
    if nxy == 2:
        in_specs.append(pl.BlockSpec((3, K2), lambda t: (0, 0)))
        args.append(ystats)
    body = functools.partial(_fused_mm_body, nxy, relu_x, relu_out)
    return pl.pallas_call(
        body,
        grid=(nt,),
        in_specs=in_specs,
        out_specs=pl.BlockSpec((MM_TILE, F), lambda t: (t, 0)),
        out_shape=jax.ShapeDtypeStruct((nt * MM_TILE, F), jnp.float32),
    )(*args)[:M]


def _mlp_body(pooled_ref, p1_ref, pb1_ref, p2_ref, pb2_ref, p3_ref, pb3_ref,
              out_ref):
    z = jnp.maximum(
        jnp.dot(pooled_ref[...], p1_ref[...],
                preferred_element_type=jnp.float32) + pb1_ref[...], 0.0)
    z = jnp.dot(z, p2_ref[...], preferred_element_type=jnp.float32) + pb2_ref[...]
    z = jnp.dot(z, p3_ref[...], preferred_element_type=jnp.float32) + pb3_ref[...]
    out_ref[...] = z


def _mlp_head(pooled, P1, pb1, P2, pb2, P3, pb3):
    return pl.pallas_call(
        _mlp_body,
        out_shape=jax.ShapeDtypeStruct((pooled.shape[0], 300), jnp.float32),
    )(pooled, P1, pb1[None, :], P2, pb2[None, :], P3, pb3[None, :])


def _pad_cols(m, k):
    return jnp.pad(m, ((0, 0), (0, k - m.shape[1])))


def kernel(x, edge_attr, params, edge_index, edge_type, edge_message_index,
           edge_message_relation, batch):
    src, dst = edge_index[0].astype(jnp.int32), edge_index[1].astype(jnp.int32)
    esrc = edge_message_index[0].astype(jnp.int32)
    edst = edge_message_index[1].astype(jnp.int32)
    erel = edge_message_relation.astype(jnp.int32)
    etype = edge_type.astype(jnp.int32)

    gidx, tile_rel, gidxD, dstD, Mp = _build_emp_layout(erel, esrc, edst)
    permE = jnp.argsort(dst).astype(jnp.int32)
    dstE = dst[permE]
    gidxR = (src * R_NODE + etype)[permE]

    DE_P = 56
    DN_P = 24
    e = _pad_cols(edge_attr, DE_P)
    h = _pad_cols(x, DN_P)

    bns = ['bn1', 'bn2', 'bn3', 'bn3', 'bn3', 'bn2']
    u = h
    ustats = _id_stats(DN_P)

    for i in range(6):
        sfx = '%d' % (i + 1)
        We_W = params['emp' + sfx + '_W']
        We_Ws = params['emp' + sfx + '_Ws']
        We_b = params['emp' + sfx + '_b']
        din = e.shape[1]
        if We_W.shape[1] != din:
            We_W = jnp.pad(We_W, ((0, 0), (0, din - We_W.shape[1]), (0, 0)))
            We_Ws = jnp.pad(We_Ws, ((0, din - We_Ws.shape[0]), (0, 0)))

        g = e[gidx]
        msgA = _bucket_mm(g, We_W, tile_rel)
        base = _fused_mm(e, We_Ws, We_b)
        agg = jax.ops.segment_sum(msgA[gidxD], dstD, num_segments=E)
        e = jnp.maximum(agg + base, 0.0)

        Wn = params['conv' + sfx + '_W']
        Wr = params['conv' + sfx + '_Wr']
        Wc = params['conv' + sfx + '_We']
        nb = params['conv' + sfx + '_b']
        dn = u.shape[1]
        if Wn.shape[1] != dn:
            Wn = jnp.pad(Wn, ((0, 0), (0, dn - Wn.shape[1]), (0, 0)))
            Wr = jnp.pad(Wr, ((0, dn - Wr.shape[0]), (0, 0)))
        Wn_cat = Wn.transpose(1, 0, 2).reshape(dn, R_NODE * H)
        relu_x = i > 0
        hW = _fused_mm(u, Wn_cat, jnp.zeros((R_NODE * H,), jnp.float32),
                       xstats=ustats, relu_x=relu_x)
        hW_flat = hW.reshape(N * R_NODE, H)
        agg_rel = jax.ops.segment_sum(hW_flat[gidxR], dstE, num_segments=N)
        aggE = jax.ops.segment_sum(e, dst, num_segments=N)
        u_next = _fused_mm(aggE, Wc, nb, y=u, b=Wr, c=agg_rel,
                           ystats=ustats, relu_x=relu_x)
        m = jnp.mean(u_next, axis=0)
        v = jnp.var(u_next, axis=0)
        gbn = params[bns[i] + '_g']
        bbn = params[bns[i] + '_beta']
        ustats = jnp.stack([m, gbn / jnp.sqrt(v + 1e-5), bbn])
        u = u_next

    h6 = (u - ustats[0][None, :]) * ustats[1][None, :] + ustats[2][None, :]
    counts = jax.ops.segment_sum(jnp.ones((N,), jnp.float32), batch,
                                 num_segments=G)
    pooled = jax.ops.segment_sum(h6, batch, num_segments=G) / \
        jnp.clip(counts, 1.0)[:, None]
    return _mlp_head(pooled, params['P1'], params['pb1'], params['P2'],
                     params['pb2'], params['P3'], params['pb3'])


# speedup vs baseline: 1.3709x; 1.0499x over previous
import functools

import jax
import jax.numpy as jnp
from jax import lax
from jax.experimental import pallas as pl
from jax.experimental.pallas import tpu as pltpu
from jax.experimental.pallas import tpu_sc as plsc

N = 10000
E = 160000
E2 = 320000
R_NODE = 7
R_EDGE = 8
G = 32
H = 512
TILE = 256
MM_TILE = 256
DE_P = 128
DN_P = 24
SEG_B = 64
MP = E2 + R_EDGE * TILE


def _ceil_to(x, m):
    return (x + m - 1) // m * m


def _build_emp_layout(rel, esrc, edst):
    M = rel.shape[0]
    perm = jnp.argsort(rel)
    esrc_s = esrc[perm].astype(jnp.int32)
    cnt = jnp.bincount(rel, length=R_EDGE).astype(jnp.int32)
    offs = jnp.concatenate([jnp.zeros(1, jnp.int32),
                            jnp.cumsum(cnt)[:-1].astype(jnp.int32)])
    cnt_pad = (cnt + TILE - 1) // TILE * TILE
    offs_pad = jnp.concatenate([jnp.zeros(1, jnp.int32),
                                jnp.cumsum(cnt_pad)[:-1].astype(jnp.int32)])
    bnd = jnp.cumsum(cnt_pad).astype(jnp.int32)
    p = jnp.arange(MP, dtype=jnp.int32)
    r_of_p = jnp.clip(jnp.searchsorted(bnd, p, side='right'),
                      0, R_EDGE - 1).astype(jnp.int32)
    k = p - offs_pad[r_of_p]
    valid = k < cnt[r_of_p]
    srcidx = jnp.minimum(offs[r_of_p] + k, M - 1)
    gidx = jnp.where(valid, esrc_s[srcidx], 0).astype(jnp.int32)
    tile_rel = r_of_p[::TILE]
    r_sorted = rel[perm]
    rank = jnp.arange(M, dtype=jnp.int32) - offs[r_sorted]
    pos = (offs_pad[r_sorted] + rank).astype(jnp.int32)
    pos_of_edge = pos[jnp.argsort(perm)]
    permD = jnp.argsort(edst)
    gidxD = pos_of_edge[permD]
    dstD = edst[permD].astype(jnp.int32)
    return gidx, tile_rel, gidxD, dstD


@functools.lru_cache(maxsize=None)
def _make_sc_gather(Ms, D, Mout):
    per = Mout // 32
    nfull = per // SEG_B
    tail = per - nfull * SEG_B
    mesh = plsc.VectorSubcoreMesh(core_axis_name="c", subcore_axis_name="s")
    scratch = [
        pltpu.VMEM((SEG_B, D), jnp.float32),
        pltpu.VMEM((SEG_B,), jnp.int32),
        pltpu.SemaphoreType.DMA,
    ]

    def body(msrc_r, g_r, out_r, stage, gv, sem):
        core = lax.axis_index("c")
        sid = lax.axis_index("s")
        wid = sid * 2 + core
        base0 = wid * per

        def _batch(j, cy):
            off = (base0 + j * SEG_B) // 8 * 8
            pltpu.sync_copy(g_r.at[pl.ds(off, SEG_B)], gv)
            pltpu.async_copy(msrc_r.at[gv], stage, sem).wait()
            pltpu.sync_copy(stage, out_r.at[pl.ds(off, SEG_B)])
            return cy
        lax.fori_loop(0, nfull, _batch, 0)
        if tail:
            off = (base0 + nfull * SEG_B) // 8 * 8
            pltpu.sync_copy(g_r.at[pl.ds(off, tail)], gv.at[pl.ds(0, tail)])
            pltpu.async_copy(msrc_r.at[gv.at[pl.ds(0, tail)]],
                             stage.at[pl.ds(0, tail)], sem).wait()
            pltpu.sync_copy(stage.at[pl.ds(0, tail)],
                            out_r.at[pl.ds(off, tail)])

    def run(msrc, gidx):
        return pl.kernel(
            body,
            out_type=jax.ShapeDtypeStruct((Mout, D), jnp.float32),
            mesh=mesh,
            scratch_types=scratch,
        )(msrc, gidx)
    return run


def _bucket_mm_body(tile_rel_ref, g_ref, w_ref, o_ref):
    r = tile_rel_ref[pl.program_id(0)]
    o_ref[...] = jnp.dot(g_ref[...], w_ref[r],
                         preferred_element_type=jnp.float32)


def _bucket_mm(g, w, tile_rel):
    Mp, K = g.shape
    F = w.shape[2]
    nt = Mp // TILE
    grid_spec = pltpu.PrefetchScalarGridSpec(
        num_scalar_prefetch=1,
        grid=(nt,),
        in_specs=[
            pl.BlockSpec((TILE, K), lambda t, sref: (t, 0)),
            pl.BlockSpec((R_EDGE, K, F), lambda t, sref: (0, 0, 0)),
        ],
        out_specs=pl.BlockSpec((TILE, F), lambda t, sref: (t, 0)),
    )
    return pl.pallas_call(
        _bucket_mm_body,
        grid_spec=grid_spec,
        out_shape=jax.ShapeDtypeStruct((Mp, F), jnp.float32),
    )(tile_rel, g, w)


def _fused_mm_body(nxy, relu_x, relu_out, *refs):
    if nxy == 2:
        x_ref, a_ref, y_ref, b_ref, c_ref, bias_ref, xs_ref, ys_ref, o_ref = refs
    else:
        x_ref, a_ref, bias_ref, xs_ref, o_ref = refs
    x = x_ref[...]
    xs = xs_ref[...]
    x = (x - xs[0:1, :]) * xs[1:2, :] + xs[2:3, :]
    if relu_x:
        x = jnp.maximum(x, 0.0)
    acc = jnp.dot(x, a_ref[...], preferred_element_type=jnp.float32)
    if nxy == 2:
        y = y_ref[...]
        ys = ys_ref[...]
        y = (y - ys[0:1, :]) * ys[1:2, :] + ys[2:3, :]
        if relu_x:
            y = jnp.maximum(y, 0.0)
        acc = acc + jnp.dot(y, b_ref[...], preferred_element_type=jnp.float32)
        acc = acc + c_ref[...]
    acc = acc + bias_ref[...]
    if relu_out:
        acc = jnp.maximum(acc, 0.0)
    o_ref[...] = acc


def _id_stats(K):
    return jnp.stack([jnp.zeros((K,), jnp.float32),
                      jnp.ones((K,), jnp.float32),
                      jnp.zeros((K,), jnp.float32)])


def _fused_mm(x, a, bias, xstats=None, y=None, b=None, c=None, ystats=None,
              relu_x=False, relu_out=False):
    M, K = x.shape
    F = a.shape[1]
    nt = _ceil_to(M, MM_TILE) // MM_TILE
    if xstats is None:
        xstats = _id_stats(K)
    nxy = 2 if y is not None else 1
    in_specs = [
        pl.BlockSpec((MM_TILE, K), lambda t: (t, 0)),
        pl.BlockSpec((K, F), lambda t: (0, 0)),
    ]
    args = [x, a]
    if nxy == 2:
        K2 = y.shape[1]
        if ystats is None:
            ystats = _id_stats(K2)
        in_specs += [
            pl.BlockSpec((MM_TILE, K2), lambda t: (t, 0)),
            pl.BlockSpec((K2, F), lambda t: (0, 0)),
            pl.BlockSpec((MM_TILE, F), lambda t: (t, 0)),
        ]
        args += [y, b, c]
    in_specs.append(pl.BlockSpec((1, F), lambda t: (0, 0)))
    args.append(bias[None, :])
    in_specs.append(pl.BlockSpec((3, K), lambda t: (0, 0)))
    args.append(xstats)
    if nxy == 2:
        in_specs.append(pl.BlockSpec((3, K2), lambda t: (0, 0)))
        args.append(ystats)
    body = functools.partial(_fused_mm_body, nxy, relu_x, relu_out)
    return pl.pallas_call(
        body,
        grid=(nt,),
        in_specs=in_specs,
        out_specs=pl.BlockSpec((MM_TILE, F), lambda t: (t, 0)),
        out_shape=jax.ShapeDtypeStruct((nt * MM_TILE, F), jnp.float32),
    )(*args)[:M]


def _mlp_body(pooled_ref, p1_ref, pb1_ref, p2_ref, pb2_ref, p3_ref, pb3_ref,
              out_ref):
    z = jnp.maximum(
        jnp.dot(pooled_ref[...], p1_ref[...],
                preferred_element_type=jnp.float32) + pb1_ref[...], 0.0)
    z = jnp.dot(z, p2_ref[...], preferred_element_type=jnp.float32) + pb2_ref[...]
    z = jnp.dot(z, p3_ref[...], preferred_element_type=jnp.float32) + pb3_ref[...]
    out_ref[...] = z


def _mlp_head(pooled, P1, pb1, P2, pb2, P3, pb3):
    return pl.pallas_call(
        _mlp_body,
        out_shape=jax.ShapeDtypeStruct((pooled.shape[0], 300), jnp.float32),
    )(pooled, P1, pb1[None, :], P2, pb2[None, :], P3, pb3[None, :])


def _pad_cols(m, k):
    return jnp.pad(m, ((0, 0), (0, k - m.shape[1])))


def kernel(x, edge_attr, params, edge_index, edge_type, edge_message_index,
           edge_message_relation, batch):
    src, dst = edge_index[0].astype(jnp.int32), edge_index[1].astype(jnp.int32)
    esrc = edge_message_index[0].astype(jnp.int32)
    edst = edge_message_index[1].astype(jnp.int32)
    erel = edge_message_relation.astype(jnp.int32)
    etype = edge_type.astype(jnp.int32)

    gidx, tile_rel, gidxD, dstD = _build_emp_layout(erel, esrc, edst)
    permE = jnp.argsort(dst).astype(jnp.int32)
    dstE = dst[permE]
    gidxR = (src * R_NODE + etype)[permE]

    e = _pad_cols(edge_attr, DE_P)
    h = _pad_cols(x, DN_P)
    gath1 = _make_sc_gather(E, DE_P, MP)
    gath = _make_sc_gather(E, H, MP)

    bns = ['bn1', 'bn2', 'bn3', 'bn3', 'bn3', 'bn2']
    u = h
    ustats = _id_stats(DN_P)

    for i in range(6):
        sfx = '%d' % (i + 1)
        We_W = params['emp' + sfx + '_W']
        We_Ws = params['emp' + sfx + '_Ws']
        We_b = params['emp' + sfx + '_b']
        din = e.shape[1]
        if We_W.shape[1] != din:
            We_W = jnp.pad(We_W, ((0, 0), (0, din - We_W.shape[1]), (0, 0)))
            We_Ws = jnp.pad(We_Ws, ((0, din - We_Ws.shape[0]), (0, 0)))

        g = (gath1 if din == DE_P else gath)(e, gidx)
        msgA = _bucket_mm(g, We_W, tile_rel)
        base = _fused_mm(e, We_Ws, We_b)
        agg = jax.ops.segment_sum(msgA[gidxD], dstD, num_segments=E)
        e = jnp.maximum(agg + base, 0.0)

        Wn = params['conv' + sfx + '_W']
        Wr = params['conv' + sfx + '_Wr']
        Wc = params['conv' + sfx + '_We']
        nb = params['conv' + sfx + '_b']
        dn = u.shape[1]
        if Wn.shape[1] != dn:
            Wn = jnp.pad(Wn, ((0, 0), (0, dn - Wn.shape[1]), (0, 0)))
            Wr = jnp.pad(Wr, ((0, dn - Wr.shape[0]), (0, 0)))
        Wn_cat = Wn.transpose(1, 0, 2).reshape(dn, R_NODE * H)
        relu_x = i > 0
        hW = _fused_mm(u, Wn_cat, jnp.zeros((R_NODE * H,), jnp.float32),
                       xstats=ustats, relu_x=relu_x)
        hW_flat = hW.reshape(N * R_NODE, H)
        agg_rel = jax.ops.segment_sum(hW_flat[gidxR], dstE, num_segments=N)
        aggE = jax.ops.segment_sum(e, dst, num_segments=N)
        u_next = _fused_mm(aggE, Wc, nb, y=u, b=Wr, c=agg_rel,
                           ystats=ustats, relu_x=relu_x)
        m = jnp.mean(u_next, axis=0)
        v = jnp.var(u_next, axis=0)
        gbn = params[bns[i] + '_g']
        bbn = params[bns[i] + '_beta']
        ustats = jnp.stack([m, gbn / jnp.sqrt(v + 1e-5), bbn])
        u = u_next

    h6 = (u - ustats[0][None, :]) * ustats[1][None, :] + ustats[2][None, :]
    counts = jax.ops.segment_sum(jnp.ones((N,), jnp.float32), batch,
                                 num_segments=G)
    pooled = jax.ops.segment_sum(h6, batch, num_segments=G) / \
        jnp.clip(counts, 1.0)[:, None]
    return _mlp_head(pooled, params['P1'], params['pb1'], params['P2'],
                     params['pb2'], params['P3'], params['pb3'])
